# baseline (speedup 1.0000x reference)
import functools
import numpy as np
import jax
import jax.numpy as jnp
from jax import lax
from jax.experimental import pallas as pl
from jax.experimental.pallas import tpu as pltpu
from jax.experimental.pallas import tpu_sc as plsc

_F32 = jnp.float32
_SC_CORES = 2
_SC_SUBCORES = 16
_SC_WORKERS = _SC_CORES * _SC_SUBCORES


def _pcall(body, **kw):
    return pl.pallas_call(body, **kw)


def _rows(HW, C):
    for r in (2048, 1024, 784, 512, 448, 392, 256, 224, 112, 98, 56, 49, 28, 16, 8):
        if HW % r == 0 and r * C * 4 <= 1 << 20:
            return r
    return HW


def _conv(x, w, groups=1):
    return lax.conv_general_dilated(
        x, w.transpose(2, 3, 1, 0), (1, 1), 'SAME',
        dimension_numbers=('NHWC', 'HWIO', 'NHWC'),
        feature_group_count=groups)


def _pixel_unshuffle(x, r):
    B, H, W, C = x.shape
    x = x.reshape(B, H // r, r, W // r, r, C)
    return x.transpose(0, 1, 3, 5, 2, 4).reshape(B, H // r, W // r, C * r * r)


def _pixel_shuffle(x, r):
    B, H, W, C = x.shape
    x = x.reshape(B, H, W, C // (r * r), r, r)
    return x.transpose(0, 1, 4, 2, 5, 3).reshape(B, H * r, W * r, C // (r * r))


def _win(t, n_win, h, w, C):
    return (t.reshape(n_win, h, n_win, w, C)
             .transpose(0, 2, 1, 3, 4)
             .reshape(n_win * n_win, h * w, C))


def _unwin(t, n_win, h, w, C):
    return (t.reshape(n_win, n_win, h, w, C)
             .transpose(0, 2, 1, 3, 4)
             .reshape(n_win * h, n_win * w, C))


def _dwconv(x, w, k, bias=None, residual=False):
    H, W, C = x.shape
    p = k // 2
    Rh = 28
    nblk = H // Rh
    wf = w.transpose(2, 3, 1, 0).reshape(k * k, C)

    def body(*refs):
        m_ref, t_ref, b_ref, w_ref = refs[0], refs[1], refs[2], refs[3]
        o_ref = refs[-1]
        i = pl.program_id(0)
        zero = jnp.zeros((p, W, C), _F32)
        top = jnp.where(i == 0, zero, t_ref[p:, :, :])
        bot = jnp.where(i == nblk - 1, zero, b_ref[:p, :, :])
        xb = jnp.concatenate([top, m_ref[...], bot], axis=0)
        zcol = jnp.zeros((Rh + 2 * p, p, C), _F32)
        xb = jnp.concatenate([zcol, xb, zcol], axis=1)
        acc = None
        for di in range(k):
            for dj in range(k):
                wt = w_ref[di * k + dj:di * k + dj + 1].reshape(1, 1, C)
                term = xb[di:di + Rh, dj:dj + W, :] * wt
                acc = term if acc is None else acc + term
        if residual:
            acc = acc + m_ref[...] + refs[4][...].reshape(1, 1, C)
        o_ref[...] = acc

    hb = Rh // (2 * p)
    in_specs = [pl.BlockSpec((Rh, W, C), lambda i: (i, 0, 0)),
                pl.BlockSpec((2 * p, W, C),
                             lambda i: (jnp.maximum(i * hb - 1, 0), 0, 0)),
                pl.BlockSpec((2 * p, W, C),
                             lambda i: (jnp.minimum((i + 1) * hb, nblk * hb - 1), 0, 0)),
                pl.BlockSpec((k * k, C), lambda i: (0, 0))]
    args = [x, x, x, wf]
    if residual:
        in_specs += [pl.BlockSpec((1, C), lambda i: (0, 0))]
        args += [bias.reshape(1, C)]
    return _pcall(
        body,
        grid=(nblk,),
        in_specs=in_specs,
        out_specs=pl.BlockSpec((Rh, W, C), lambda i: (i, 0, 0)),
        out_shape=jax.ShapeDtypeStruct((H, W, C), _F32),
    )(*args)


def _ln_qv(x, g, b, wq, wv, bq, bv):
    HW, C = x.shape
    R = _rows(HW, C)

    def body(x_ref, g_ref, b_ref, wq_ref, wv_ref, bq_ref, bv_ref,
             xn_ref, q_ref, v_ref):
        x_ = x_ref[...]
        mu = jnp.mean(x_, axis=-1, keepdims=True)
        xc = x_ - mu
        var = jnp.mean(xc * xc, axis=-1, keepdims=True)
        xn = xc * lax.rsqrt(var + 1e-5) * g_ref[...] + b_ref[...]
        xn_ref[...] = xn
        q_ref[...] = jnp.dot(xn, wq_ref[...], preferred_element_type=_F32) + bq_ref[...]
        v_ref[...] = jnp.dot(xn, wv_ref[...], preferred_element_type=_F32) + bv_ref[...]

    full = lambda shape: pl.BlockSpec(shape, lambda i: (0,) * len(shape))
    row = pl.BlockSpec((R, C), lambda i: (i, 0))
    return _pcall(
        body,
        grid=(HW // R,),
        in_specs=[row, full((1, C)), full((1, C)), full((C, C)), full((C, C)),
                  full((1, C)), full((1, C))],
        out_specs=[row, row, row],
        out_shape=[jax.ShapeDtypeStruct((HW, C), _F32)] * 3,
    )(x, g.reshape(1, C), b.reshape(1, C), wq, wv,
      bq.reshape(1, C), bv.reshape(1, C))


def _route_proj(xp, xm, wq, wk, wv, bq, bk, bv, topk):
    Pm, C = xp.shape
    P = xm.shape[0]

    Cpad = -(-C // 128) * 128

    def body(xp_ref, xm_ref, wq_ref, wk_ref, wv_ref, bq_ref, bk_ref, bv_ref,
             kp_ref, vp_ref, idx_ref):
        xp_ = xp_ref[...]
        xm_ = xm_ref[...]
        kp = jnp.dot(xp_, wk_ref[...], preferred_element_type=_F32) + bk_ref[...]
        vp = jnp.dot(xp_, wv_ref[...], preferred_element_type=_F32) + bv_ref[...]
        if Cpad > C:
            z = jnp.zeros((Pm, Cpad - C), _F32)
            kp = jnp.concatenate([kp, z], axis=1)
            vp = jnp.concatenate([vp, z], axis=1)
        kp_ref[...] = kp
        vp_ref[...] = vp
        qm = jnp.dot(xm_, wq_ref[...], preferred_element_type=_F32) + bq_ref[...]
        km = jnp.dot(xm_, wk_ref[...], preferred_element_type=_F32) + bk_ref[...]
        a = lax.dot_general(qm, km, (((1,), (1,)), ((), ())),
                            preferred_element_type=_F32)
        col = lax.broadcasted_iota(jnp.int32, (P, P), 1)
        cols = []
        for _ in range(topk):
            mx = jnp.max(a, axis=1, keepdims=True)
            am = jnp.min(jnp.where(a >= mx, col, P), axis=1, keepdims=True)
            cols.append(am)
            a = jnp.where(col == am, -jnp.inf, a)
        idx_ref[...] = jnp.concatenate(cols, axis=1)

    return _pcall(
        body,
        out_shape=[jax.ShapeDtypeStruct((Pm, Cpad), _F32),
                   jax.ShapeDtypeStruct((Pm, Cpad), _F32),
                   jax.ShapeDtypeStruct((P, topk), jnp.int32)],
    )(xp, xm, wq, wk, wv, bq.reshape(1, C), bk.reshape(1, C), bv.reshape(1, C))


def _sc_gather(kp, vp, idxr):
    N, C = kp.shape
    Npad = idxr.shape[0]
    b = Npad // _SC_WORKERS
    mesh = plsc.VectorSubcoreMesh(core_axis_name="c", subcore_axis_name="s")

    @functools.partial(
        pl.kernel, mesh=mesh,
        out_type=[jax.ShapeDtypeStruct((Npad, C), _F32),
                  jax.ShapeDtypeStruct((Npad, C), _F32)],
        scratch_types=[pltpu.VMEM((b,), jnp.int32),
                       pltpu.VMEM((b, C), _F32),
                       pltpu.VMEM((b, C), _F32),
                       pltpu.SemaphoreType.DMA,
                       pltpu.SemaphoreType.DMA],
    )
    def gk(kp_hbm, vp_hbm, idx_hbm, ko_hbm, vo_hbm, idx_v, kr, vr, s1, s2):
        wid = jax.lax.axis_index("s") * _SC_CORES + jax.lax.axis_index("c")
        base = wid * b
        pltpu.sync_copy(idx_hbm.at[pl.ds(base, b)], idx_v)
        for off in range(0, b, 128):
            n = min(128, b - off)
            c1 = pltpu.async_copy(kp_hbm.at[idx_v.at[pl.ds(off, n)]],
                                  kr.at[pl.ds(off, n)], s1)
            c2 = pltpu.async_copy(vp_hbm.at[idx_v.at[pl.ds(off, n)]],
                                  vr.at[pl.ds(off, n)], s2)
            c1.wait()
            c2.wait()
        pltpu.sync_copy(kr, ko_hbm.at[pl.ds(base, b)])
        pltpu.sync_copy(vr, vo_hbm.at[pl.ds(base, b)])

    return gk(kp, vp, idxr)


def _attn(qw, ks, vs, nh, kvtot):
    P, hw, C = qw.shape
    Cp = ks.shape[2]
    c = C // nh
    scale = c ** -0.5
    G = 14 if P % 14 == 0 else (7 if P % 7 == 0 else 1)

    def body(qw_ref, ks_ref, vs_ref, ow_ref):
        for g in range(G):
            q = qw_ref[g]
            ks_ = ks_ref[g]
            vs_ = vs_ref[g]
            outs = []
            for hh in range(nh):
                qh = q[:, hh * c:(hh + 1) * c] * scale
                kh = ks_[:, hh * c:(hh + 1) * c]
                vh = vs_[:, hh * c:(hh + 1) * c]
                logits = lax.dot_general(qh, kh, (((1,), (1,)), ((), ())),
                                         preferred_element_type=_F32)
                pa = jax.nn.softmax(logits, axis=-1)
                outs.append(jnp.dot(pa, vh, preferred_element_type=_F32))
            ow_ref[g] = jnp.concatenate(outs, 1) if nh > 1 else outs[0]

    return _pcall(
        body,
        grid=(P // G,),
        in_specs=[pl.BlockSpec((G, hw, C), lambda i: (i, 0, 0)),
                  pl.BlockSpec((G, kvtot, Cp), lambda i: (i, 0, 0)),
                  pl.BlockSpec((G, kvtot, Cp), lambda i: (i, 0, 0))],
        out_specs=pl.BlockSpec((G, hw, C), lambda i: (i, 0, 0)),
        out_shape=jax.ShapeDtypeStruct((P, hw, C), _F32),
    )(qw, ks, vs)


def _wo_mlp(xres, aw, lepe, wo, wob, g, b, w1, b1, w2, b2):
    HW, C = xres.shape
    C2 = w1.shape[1]
    R = _rows(HW, C)

    def body(x_ref, a_ref, l_ref, wo_ref, wob_ref, g_ref, b_ref,
             w1_ref, b1_ref, w2_ref, b2_ref, o_ref):
        y = a_ref[...] + l_ref[...]
        x1 = x_ref[...] + jnp.dot(y, wo_ref[...],
                                  preferred_element_type=_F32) + wob_ref[...]
        mu = jnp.mean(x1, axis=-1, keepdims=True)
        xc = x1 - mu
        var = jnp.mean(xc * xc, axis=-1, keepdims=True)
        xn = xc * lax.rsqrt(var + 1e-5) * g_ref[...] + b_ref[...]
        h = jax.nn.gelu(jnp.dot(xn, w1_ref[...], preferred_element_type=_F32)
                        + b1_ref[...])
        o_ref[...] = x1 + jnp.dot(h, w2_ref[...],
                                  preferred_element_type=_F32) + b2_ref[...]

    full = lambda shape: pl.BlockSpec(shape, lambda i: (0,) * len(shape))
    row = pl.BlockSpec((R, C), lambda i: (i, 0))
    return _pcall(
        body,
        grid=(HW // R,),
        in_specs=[row, row, row, full((C, C)), full((1, C)), full((1, C)),
                  full((1, C)), full((C, C2)), full((1, C2)), full((C2, C)),
                  full((1, C))],
        out_specs=row,
        out_shape=jax.ShapeDtypeStruct((HW, C), _F32),
    )(xres, aw, lepe, wo, wob.reshape(1, C), g.reshape(1, C), b.reshape(1, C),
      w1, b1.reshape(1, C2), w2, b2.reshape(1, C))


def _block(x, p, n_win, nh, topk, kv_per_win):
    C = x.shape[-1]
    x = _dwconv(x[0], p['cpe_w'], 3, bias=p['cpe_b'], residual=True)[None]
    _, H, W, _ = x.shape
    h, w = H // n_win, W // n_win
    P = n_win * n_win
    m = kv_per_win * kv_per_win
    bh, bw = h // kv_per_win, w // kv_per_win
    HW = H * W

    wq, wk, wv = jnp.split(p['qkv_w'], 3, axis=1)
    bq, bk, bv = jnp.split(p['qkv_b'], 3)

    x2 = x[0]
    xn, q, v = _ln_qv(x2.reshape(HW, C), p['ln1_g'], p['ln1_b'],
                      wq, wv, bq, bv)

    xn3 = xn.reshape(H, W, C)
    xp = (xn3.reshape(n_win, kv_per_win, bh, n_win, kv_per_win, bw, C)
             .mean(axis=(2, 5))
             .transpose(0, 2, 1, 3, 4)
             .reshape(P * m, C))
    xm = xn3.reshape(n_win, h, n_win, w, C).mean(axis=(1, 3)).reshape(P, C)

    kp, vp, idx = _route_proj(xp, xm, wq, wk, wv, bq, bk, bv, topk)

    kvtot = topk * m
    idxe = (idx[:, :, None] * m + jnp.arange(m, dtype=jnp.int32)).reshape(-1)
    npad = -(P * kvtot) // 256 * -256
    idxr = jnp.zeros((npad,), jnp.int32).at[:P * kvtot].set(idxe)
    ks, vs = _sc_gather(kp, vp, idxr)
    Cpad = kp.shape[1]
    ks = ks[:P * kvtot].reshape(P, kvtot, Cpad)
    vs = vs[:P * kvtot].reshape(P, kvtot, Cpad)

    qw = _win(q.reshape(H, W, C), n_win, h, w, C)
    aw = _attn(qw, ks, vs, nh, kvtot)
    a_spat = _unwin(aw, n_win, h, w, C).reshape(HW, C)

    lepe = _dwconv(v.reshape(H, W, C), p['lepe_w'], 5).reshape(HW, C)

    out = _wo_mlp(x2.reshape(HW, C), a_spat, lepe, p['wo_w'], p['wo_b'],
                  p['ln2_g'], p['ln2_b'], p['mlp_w1'], p['mlp_b1'],
                  p['mlp_w2'], p['mlp_b2'])
    return out.reshape(1, H, W, C)


def kernel(img0, img1, warped_img0, warped_img1, mask, flow, c0_0, c0_1,
           c0_2, c0_3, c1_0, c1_1, c1_2, c1_3, mask_guide_0, mask_guide_1,
           mask_guide_2, params):
    p = params
    nhwc = lambda t: t.transpose(0, 2, 3, 1)
    inp = jnp.concatenate([img0, img1, mask, mask_guide_0, warped_img0,
                           warped_img1, c0_0, c1_0, flow], 1)
    x1 = _conv(nhwc(inp), p['patch_embed'])
    for bp in p['enc1']:
        x1 = _block(x1, bp, 14, 1, 6, 2)
    f1 = _conv(nhwc(jnp.concatenate([mask_guide_1, c0_1, c1_1], 1)),
               p['c_down1'])
    x2 = _pixel_unshuffle(_conv(x1, p['down1_2']), 2)
    x2 = jnp.concatenate([x2, f1], -1)
    for bp in p['enc2']:
        x2 = _block(x2, bp, 7, 2, 4, 1)
    f2 = _conv(nhwc(jnp.concatenate([mask_guide_2, c0_2, c1_2], 1)),
               p['c_down2'])
    x3 = _pixel_unshuffle(_conv(x2, p['down2_3']), 2)
    x3 = jnp.concatenate([x3, f2], -1)
    for bp in p['enc3']:
        x3 = _block(x3, bp, 7, 4, 4, 1)
    f3 = _conv(nhwc(jnp.concatenate([c0_3, c1_3], 1)), p['c_down3'])
    x4 = _pixel_unshuffle(_conv(x3, p['down3_4']), 2)
    x4 = jnp.concatenate([x4, f3], -1)
    for bp in p['latent']:
        x4 = _block(x4, bp, 7, 8, 4, 1)
    d3 = _pixel_shuffle(_conv(x4, p['up4_3']), 2)
    d3 = _conv(jnp.concatenate([d3, x3], -1), p['reduce3'])
    for bp in p['dec3']:
        d3 = _block(d3, bp, 7, 1, 4, 1)
    d2 = _pixel_shuffle(_conv(d3, p['up3_2']), 2)
    d2 = _conv(jnp.concatenate([d2, x2], -1), p['reduce2'])
    for bp in p['dec2']:
        d2 = _block(d2, bp, 7, 1, 4, 1)
    d1 = _pixel_shuffle(_conv(d2, p['up2_1']), 2)
    d1 = jnp.concatenate([d1, x1], -1)
    for bp in p['dec1']:
        d1 = _block(d1, bp, 7, 1, 4, 2)
    for bp in p['refine']:
        d1 = _block(d1, bp, 14, 1, 6, 2)
    return jax.nn.sigmoid(_conv(d1, p['out_w'])).transpose(0, 3, 1, 2)


# speedup vs baseline: 1.0605x; 1.0605x over previous
import functools
import numpy as np
import jax
import jax.numpy as jnp
from jax import lax
from jax.experimental import pallas as pl
from jax.experimental.pallas import tpu as pltpu
from jax.experimental.pallas import tpu_sc as plsc

_F32 = jnp.float32
_SC_CORES = 2
_SC_SUBCORES = 16
_SC_WORKERS = _SC_CORES * _SC_SUBCORES


def _pcall(body, **kw):
    return pl.pallas_call(body, **kw)


def _rows(HW, C):
    for r in (2048, 1024, 784, 512, 448, 392, 256, 224, 112, 98, 56, 49, 28, 16, 8):
        if HW % r == 0 and r * C * 4 <= 1 << 20:
            return r
    return HW


def _conv(x, w, groups=1):
    return lax.conv_general_dilated(
        x, w.transpose(2, 3, 1, 0), (1, 1), 'SAME',
        dimension_numbers=('NHWC', 'HWIO', 'NHWC'),
        feature_group_count=groups)


def _pixel_unshuffle(x, r):
    B, H, W, C = x.shape
    x = x.reshape(B, H // r, r, W // r, r, C)
    return x.transpose(0, 1, 3, 5, 2, 4).reshape(B, H // r, W // r, C * r * r)


def _pixel_shuffle(x, r):
    B, H, W, C = x.shape
    x = x.reshape(B, H, W, C // (r * r), r, r)
    return x.transpose(0, 1, 4, 2, 5, 3).reshape(B, H * r, W * r, C // (r * r))


def _win(t, n_win, h, w, C):
    return (t.reshape(n_win, h, n_win, w, C)
             .transpose(0, 2, 1, 3, 4)
             .reshape(n_win * n_win, h * w, C))


def _unwin(t, n_win, h, w, C):
    return (t.reshape(n_win, n_win, h, w, C)
             .transpose(0, 2, 1, 3, 4)
             .reshape(n_win * h, n_win * w, C))


def _dwconv(x, w, k, bias=None, residual=False):
    H, W, C = x.shape
    p = k // 2
    Rh = 28
    nblk = H // Rh
    wf = w.transpose(2, 3, 1, 0).reshape(k * k, C)

    def body(*refs):
        m_ref, t_ref, b_ref, w_ref = refs[0], refs[1], refs[2], refs[3]
        o_ref = refs[-1]
        i = pl.program_id(0)
        zero = jnp.zeros((p, W, C), _F32)
        top = jnp.where(i == 0, zero, t_ref[p:, :, :])
        bot = jnp.where(i == nblk - 1, zero, b_ref[:p, :, :])
        xb = jnp.concatenate([top, m_ref[...], bot], axis=0)
        zcol = jnp.zeros((Rh + 2 * p, p, C), _F32)
        xb = jnp.concatenate([zcol, xb, zcol], axis=1)
        acc = None
        for di in range(k):
            for dj in range(k):
                wt = w_ref[di * k + dj:di * k + dj + 1].reshape(1, 1, C)
                term = xb[di:di + Rh, dj:dj + W, :] * wt
                acc = term if acc is None else acc + term
        if residual:
            acc = acc + m_ref[...] + refs[4][...].reshape(1, 1, C)
        o_ref[...] = acc

    hb = Rh // (2 * p)
    in_specs = [pl.BlockSpec((Rh, W, C), lambda i: (i, 0, 0)),
                pl.BlockSpec((2 * p, W, C),
                             lambda i: (jnp.maximum(i * hb - 1, 0), 0, 0)),
                pl.BlockSpec((2 * p, W, C),
                             lambda i: (jnp.minimum((i + 1) * hb, nblk * hb - 1), 0, 0)),
                pl.BlockSpec((k * k, C), lambda i: (0, 0))]
    args = [x, x, x, wf]
    if residual:
        in_specs += [pl.BlockSpec((1, C), lambda i: (0, 0))]
        args += [bias.reshape(1, C)]
    return _pcall(
        body,
        grid=(nblk,),
        in_specs=in_specs,
        out_specs=pl.BlockSpec((Rh, W, C), lambda i: (i, 0, 0)),
        out_shape=jax.ShapeDtypeStruct((H, W, C), _F32),
    )(*args)


@functools.lru_cache(maxsize=None)
def _pool_rows_mat(bh, bw, W, Kw):
    pm = np.zeros((Kw, bh * W), np.float32)
    for u in range(Kw):
        for r in range(bh):
            for c in range(bw):
                pm[u, r * W + u * bw + c] = 1.0 / (bh * bw)
    return jnp.asarray(pm)


@functools.lru_cache(maxsize=None)
def _win_avg_mat(n_win, kv):
    P = n_win * n_win
    m = kv * kv
    Kw = n_win * kv
    av = np.zeros((P, P * m), np.float32)
    for wi in range(n_win):
        for wj in range(n_win):
            for ki in range(kv):
                for kj in range(kv):
                    av[wi * n_win + wj, (wi * kv + ki) * Kw + wj * kv + kj] = 1.0 / m
    return jnp.asarray(av)


def _ln_qv_pool(x, g, b, wq, wv, bq, bv, pmx, Kh, Kw):
    HW, C = x.shape
    R = HW // Kh

    def body(x_ref, g_ref, b_ref, wq_ref, wv_ref, bq_ref, bv_ref, pm_ref,
             q_ref, v_ref, xp_ref):
        x_ = x_ref[...]
        mu = jnp.mean(x_, axis=-1, keepdims=True)
        xc = x_ - mu
        var = jnp.mean(xc * xc, axis=-1, keepdims=True)
        xn = xc * lax.rsqrt(var + 1e-5) * g_ref[...] + b_ref[...]
        q_ref[...] = jnp.dot(xn, wq_ref[...], preferred_element_type=_F32) + bq_ref[...]
        v_ref[...] = jnp.dot(xn, wv_ref[...], preferred_element_type=_F32) + bv_ref[...]
        xp_ref[0] = jnp.dot(pm_ref[...], xn, preferred_element_type=_F32)

    full = lambda shape: pl.BlockSpec(shape, lambda i: (0,) * len(shape))
    row = pl.BlockSpec((R, C), lambda i: (i, 0))
    return _pcall(
        body,
        grid=(Kh,),
        in_specs=[row, full((1, C)), full((1, C)), full((C, C)), full((C, C)),
                  full((1, C)), full((1, C)), full((Kw, R))],
        out_specs=[row, row, pl.BlockSpec((1, Kw, C), lambda i: (i, 0, 0))],
        out_shape=[jax.ShapeDtypeStruct((HW, C), _F32),
                   jax.ShapeDtypeStruct((HW, C), _F32),
                   jax.ShapeDtypeStruct((Kh, Kw, C), _F32)],
    )(x, g.reshape(1, C), b.reshape(1, C), wq, wv,
      bq.reshape(1, C), bv.reshape(1, C), pmx)


def _route_proj(xp, avg, P, wq, wk, wv, bq, bk, bv, topk):
    Pm, C = xp.shape

    Cpad = -(-C // 128) * 128

    def body(*refs):
        if avg is None:
            (xp_ref, wq_ref, wk_ref, wv_ref, bq_ref, bk_ref, bv_ref,
             kp_ref, vp_ref, idx_ref) = refs
        else:
            (xp_ref, avg_ref, wq_ref, wk_ref, wv_ref, bq_ref, bk_ref, bv_ref,
             kp_ref, vp_ref, idx_ref) = refs
        xp_ = xp_ref[...]
        kp = jnp.dot(xp_, wk_ref[...], preferred_element_type=_F32) + bk_ref[...]
        vp = jnp.dot(xp_, wv_ref[...], preferred_element_type=_F32) + bv_ref[...]
        if Cpad > C:
            z = jnp.zeros((Pm, Cpad - C), _F32)
            kp = jnp.concatenate([kp, z], axis=1)
            vp = jnp.concatenate([vp, z], axis=1)
        kp_ref[...] = kp
        vp_ref[...] = vp
        xm_ = xp_ if avg is None else jnp.dot(avg_ref[...], xp_,
                                              preferred_element_type=_F32)
        qm = jnp.dot(xm_, wq_ref[...], preferred_element_type=_F32) + bq_ref[...]
        km = jnp.dot(xm_, wk_ref[...], preferred_element_type=_F32) + bk_ref[...]
        a = lax.dot_general(qm, km, (((1,), (1,)), ((), ())),
                            preferred_element_type=_F32)
        col = lax.broadcasted_iota(jnp.int32, (P, P), 1)
        cols = []
        for _ in range(topk):
            mx = jnp.max(a, axis=1, keepdims=True)
            am = jnp.min(jnp.where(a >= mx, col, P), axis=1, keepdims=True)
            cols.append(am)
            a = jnp.where(col == am, -jnp.inf, a)
        idx_ref[...] = jnp.concatenate(cols, axis=1)

    args = [xp] + ([] if avg is None else [avg]) + [
        wq, wk, wv, bq.reshape(1, C), bk.reshape(1, C), bv.reshape(1, C)]
    return _pcall(
        body,
        out_shape=[jax.ShapeDtypeStruct((Pm, Cpad), _F32),
                   jax.ShapeDtypeStruct((Pm, Cpad), _F32),
                   jax.ShapeDtypeStruct((P, topk), jnp.int32)],
    )(*args)


def _sc_gather(kp, vp, idxr):
    N, C = kp.shape
    Npad = idxr.shape[0]
    b = Npad // _SC_WORKERS
    mesh = plsc.VectorSubcoreMesh(core_axis_name="c", subcore_axis_name="s")

    @functools.partial(
        pl.kernel, mesh=mesh,
        out_type=[jax.ShapeDtypeStruct((Npad, C), _F32),
                  jax.ShapeDtypeStruct((Npad, C), _F32)],
        scratch_types=[pltpu.VMEM((b,), jnp.int32),
                       pltpu.VMEM((b, C), _F32),
                       pltpu.VMEM((b, C), _F32),
                       pltpu.SemaphoreType.DMA,
                       pltpu.SemaphoreType.DMA],
    )
    def gk(kp_hbm, vp_hbm, idx_hbm, ko_hbm, vo_hbm, idx_v, kr, vr, s1, s2):
        wid = jax.lax.axis_index("s") * _SC_CORES + jax.lax.axis_index("c")
        base = wid * b
        pltpu.sync_copy(idx_hbm.at[pl.ds(base, b)], idx_v)
        for off in range(0, b, 128):
            n = min(128, b - off)
            c1 = pltpu.async_copy(kp_hbm.at[idx_v.at[pl.ds(off, n)]],
                                  kr.at[pl.ds(off, n)], s1)
            c2 = pltpu.async_copy(vp_hbm.at[idx_v.at[pl.ds(off, n)]],
                                  vr.at[pl.ds(off, n)], s2)
            c1.wait()
            c2.wait()
        pltpu.sync_copy(kr, ko_hbm.at[pl.ds(base, b)])
        pltpu.sync_copy(vr, vo_hbm.at[pl.ds(base, b)])

    return gk(kp, vp, idxr)


def _attn(qw, ks, vs, nh, kvtot):
    P, hw, C = qw.shape
    Cp = ks.shape[2]
    c = C // nh
    scale = c ** -0.5
    G = 14 if P % 14 == 0 else (7 if P % 7 == 0 else 1)

    def body(qw_ref, ks_ref, vs_ref, ow_ref):
        for g in range(G):
            q = qw_ref[g]
            ks_ = ks_ref[g]
            vs_ = vs_ref[g]
            outs = []
            for hh in range(nh):
                qh = q[:, hh * c:(hh + 1) * c] * scale
                kh = ks_[:, hh * c:(hh + 1) * c]
                vh = vs_[:, hh * c:(hh + 1) * c]
                logits = lax.dot_general(qh, kh, (((1,), (1,)), ((), ())),
                                         preferred_element_type=_F32)
                pa = jax.nn.softmax(logits, axis=-1)
                outs.append(jnp.dot(pa, vh, preferred_element_type=_F32))
            ow_ref[g] = jnp.concatenate(outs, 1) if nh > 1 else outs[0]

    return _pcall(
        body,
        grid=(P // G,),
        in_specs=[pl.BlockSpec((G, hw, C), lambda i: (i, 0, 0)),
                  pl.BlockSpec((G, kvtot, Cp), lambda i: (i, 0, 0)),
                  pl.BlockSpec((G, kvtot, Cp), lambda i: (i, 0, 0))],
        out_specs=pl.BlockSpec((G, hw, C), lambda i: (i, 0, 0)),
        out_shape=jax.ShapeDtypeStruct((P, hw, C), _F32),
    )(qw, ks, vs)


def _wo_mlp(xres, aw, lepe, wo, wob, g, b, w1, b1, w2, b2):
    HW, C = xres.shape
    C2 = w1.shape[1]
    R = _rows(HW, C)

    def body(x_ref, a_ref, l_ref, wo_ref, wob_ref, g_ref, b_ref,
             w1_ref, b1_ref, w2_ref, b2_ref, o_ref):
        y = a_ref[...] + l_ref[...]
        x1 = x_ref[...] + jnp.dot(y, wo_ref[...],
                                  preferred_element_type=_F32) + wob_ref[...]
        mu = jnp.mean(x1, axis=-1, keepdims=True)
        xc = x1 - mu
        var = jnp.mean(xc * xc, axis=-1, keepdims=True)
        xn = xc * lax.rsqrt(var + 1e-5) * g_ref[...] + b_ref[...]
        h = jax.nn.gelu(jnp.dot(xn, w1_ref[...], preferred_element_type=_F32)
                        + b1_ref[...])
        o_ref[...] = x1 + jnp.dot(h, w2_ref[...],
                                  preferred_element_type=_F32) + b2_ref[...]

    full = lambda shape: pl.BlockSpec(shape, lambda i: (0,) * len(shape))
    row = pl.BlockSpec((R, C), lambda i: (i, 0))
    return _pcall(
        body,
        grid=(HW // R,),
        in_specs=[row, row, row, full((C, C)), full((1, C)), full((1, C)),
                  full((1, C)), full((C, C2)), full((1, C2)), full((C2, C)),
                  full((1, C))],
        out_specs=row,
        out_shape=jax.ShapeDtypeStruct((HW, C), _F32),
    )(xres, aw, lepe, wo, wob.reshape(1, C), g.reshape(1, C), b.reshape(1, C),
      w1, b1.reshape(1, C2), w2, b2.reshape(1, C))


def _block(x, p, n_win, nh, topk, kv_per_win):
    C = x.shape[-1]
    x = _dwconv(x[0], p['cpe_w'], 3, bias=p['cpe_b'], residual=True)[None]
    _, H, W, _ = x.shape
    h, w = H // n_win, W // n_win
    P = n_win * n_win
    m = kv_per_win * kv_per_win
    bh, bw = h // kv_per_win, w // kv_per_win
    HW = H * W

    wq, wk, wv = jnp.split(p['qkv_w'], 3, axis=1)
    bq, bk, bv = jnp.split(p['qkv_b'], 3)

    Kh = Kw = n_win * kv_per_win
    x2 = x[0]
    pmx = _pool_rows_mat(bh, bw, W, Kw)
    q, v, xp3 = _ln_qv_pool(x2.reshape(HW, C), p['ln1_g'], p['ln1_b'],
                            wq, wv, bq, bv, pmx, Kh, Kw)
    xp = xp3.reshape(Kh * Kw, C)

    avg = _win_avg_mat(n_win, kv_per_win) if kv_per_win > 1 else None
    kp, vp, idx = _route_proj(xp, avg, P, wq, wk, wv, bq, bk, bv, topk)

    kvtot = topk * m
    kv_ = kv_per_win
    wi = idx // n_win
    wj = idx % n_win
    ki = jnp.arange(kv_, dtype=jnp.int32).reshape(1, 1, kv_, 1)
    kj = jnp.arange(kv_, dtype=jnp.int32).reshape(1, 1, 1, kv_)
    rows = ((wi[:, :, None, None] * kv_ + ki) * Kw
            + wj[:, :, None, None] * kv_ + kj).reshape(-1)
    npad = -(P * kvtot) // 256 * -256
    idxr = jnp.zeros((npad,), jnp.int32).at[:P * kvtot].set(rows)

    lepe = _dwconv(v.reshape(H, W, C), p['lepe_w'], 5).reshape(HW, C)
    qw = _win(q.reshape(H, W, C), n_win, h, w, C)

    ks, vs = _sc_gather(kp, vp, idxr)
    Cpad = kp.shape[1]
    ks = ks[:P * kvtot].reshape(P, kvtot, Cpad)
    vs = vs[:P * kvtot].reshape(P, kvtot, Cpad)

    aw = _attn(qw, ks, vs, nh, kvtot)
    a_spat = _unwin(aw, n_win, h, w, C).reshape(HW, C)

    out = _wo_mlp(x2.reshape(HW, C), a_spat, lepe, p['wo_w'], p['wo_b'],
                  p['ln2_g'], p['ln2_b'], p['mlp_w1'], p['mlp_b1'],
                  p['mlp_w2'], p['mlp_b2'])
    return out.reshape(1, H, W, C)


def kernel(img0, img1, warped_img0, warped_img1, mask, flow, c0_0, c0_1,
           c0_2, c0_3, c1_0, c1_1, c1_2, c1_3, mask_guide_0, mask_guide_1,
           mask_guide_2, params):
    p = params
    nhwc = lambda t: t.transpose(0, 2, 3, 1)
    inp = jnp.concatenate([img0, img1, mask, mask_guide_0, warped_img0,
                           warped_img1, c0_0, c1_0, flow], 1)
    x1 = _conv(nhwc(inp), p['patch_embed'])
    for bp in p['enc1']:
        x1 = _block(x1, bp, 14, 1, 6, 2)
    f1 = _conv(nhwc(jnp.concatenate([mask_guide_1, c0_1, c1_1], 1)),
               p['c_down1'])
    x2 = _pixel_unshuffle(_conv(x1, p['down1_2']), 2)
    x2 = jnp.concatenate([x2, f1], -1)
    for bp in p['enc2']:
        x2 = _block(x2, bp, 7, 2, 4, 1)
    f2 = _conv(nhwc(jnp.concatenate([mask_guide_2, c0_2, c1_2], 1)),
               p['c_down2'])
    x3 = _pixel_unshuffle(_conv(x2, p['down2_3']), 2)
    x3 = jnp.concatenate([x3, f2], -1)
    for bp in p['enc3']:
        x3 = _block(x3, bp, 7, 4, 4, 1)
    f3 = _conv(nhwc(jnp.concatenate([c0_3, c1_3], 1)), p['c_down3'])
    x4 = _pixel_unshuffle(_conv(x3, p['down3_4']), 2)
    x4 = jnp.concatenate([x4, f3], -1)
    for bp in p['latent']:
        x4 = _block(x4, bp, 7, 8, 4, 1)
    d3 = _pixel_shuffle(_conv(x4, p['up4_3']), 2)
    d3 = _conv(jnp.concatenate([d3, x3], -1), p['reduce3'])
    for bp in p['dec3']:
        d3 = _block(d3, bp, 7, 1, 4, 1)
    d2 = _pixel_shuffle(_conv(d3, p['up3_2']), 2)
    d2 = _conv(jnp.concatenate([d2, x2], -1), p['reduce2'])
    for bp in p['dec2']:
        d2 = _block(d2, bp, 7, 1, 4, 1)
    d1 = _pixel_shuffle(_conv(d2, p['up2_1']), 2)
    d1 = jnp.concatenate([d1, x1], -1)
    for bp in p['dec1']:
        d1 = _block(d1, bp, 7, 1, 4, 2)
    for bp in p['refine']:
        d1 = _block(d1, bp, 14, 1, 6, 2)
    return jax.nn.sigmoid(_conv(d1, p['out_w'])).transpose(0, 3, 1, 2)


# speedup vs baseline: 1.0629x; 1.0023x over previous
import functools
import numpy as np
import jax
import jax.numpy as jnp
from jax import lax
from jax.experimental import pallas as pl
from jax.experimental.pallas import tpu as pltpu
from jax.experimental.pallas import tpu_sc as plsc

_F32 = jnp.float32
_SC_CORES = 2
_SC_SUBCORES = 16
_SC_WORKERS = _SC_CORES * _SC_SUBCORES


def _pcall(body, **kw):
    return pl.pallas_call(body, **kw)


def _rows(HW, C):
    for r in (2048, 1024, 784, 512, 448, 392, 256, 224, 112, 98, 56, 49, 28, 16, 8):
        if HW % r == 0 and r * C * 4 <= 1 << 20:
            return r
    return HW


def _conv(x, w, groups=1):
    return lax.conv_general_dilated(
        x, w.transpose(2, 3, 1, 0), (1, 1), 'SAME',
        dimension_numbers=('NHWC', 'HWIO', 'NHWC'),
        feature_group_count=groups)


def _pixel_unshuffle(x, r):
    B, H, W, C = x.shape
    x = x.reshape(B, H // r, r, W // r, r, C)
    return x.transpose(0, 1, 3, 5, 2, 4).reshape(B, H // r, W // r, C * r * r)


def _pixel_shuffle(x, r):
    B, H, W, C = x.shape
    x = x.reshape(B, H, W, C // (r * r), r, r)
    return x.transpose(0, 1, 4, 2, 5, 3).reshape(B, H * r, W * r, C // (r * r))


def _win(t, n_win, h, w, C):
    return (t.reshape(n_win, h, n_win, w, C)
             .transpose(0, 2, 1, 3, 4)
             .reshape(n_win * n_win, h * w, C))


def _unwin(t, n_win, h, w, C):
    return (t.reshape(n_win, n_win, h, w, C)
             .transpose(0, 2, 1, 3, 4)
             .reshape(n_win * h, n_win * w, C))


def _dwconv(x, w, k, bias=None, residual=False):
    H, W, C = x.shape
    p = k // 2
    Rh = 28
    nblk = H // Rh
    wf = w.transpose(2, 3, 1, 0).reshape(k * k, C)

    def body(*refs):
        m_ref, t_ref, b_ref, w_ref = refs[0], refs[1], refs[2], refs[3]
        o_ref = refs[-1]
        i = pl.program_id(0)
        zero = jnp.zeros((p, W, C), _F32)
        top = jnp.where(i == 0, zero, t_ref[p:, :, :])
        bot = jnp.where(i == nblk - 1, zero, b_ref[:p, :, :])
        xb = jnp.concatenate([top, m_ref[...], bot], axis=0)
        zcol = jnp.zeros((Rh + 2 * p, p, C), _F32)
        xb = jnp.concatenate([zcol, xb, zcol], axis=1)
        acc = None
        for dj in range(k):
            xj = xb[:, dj:dj + W, :]
            for di in range(k):
                wt = w_ref[di * k + dj:di * k + dj + 1].reshape(1, 1, C)
                term = xj[di:di + Rh] * wt
                acc = term if acc is None else acc + term
        if residual:
            acc = acc + m_ref[...] + refs[4][...].reshape(1, 1, C)
        o_ref[...] = acc

    hb = Rh // (2 * p)
    in_specs = [pl.BlockSpec((Rh, W, C), lambda i: (i, 0, 0)),
                pl.BlockSpec((2 * p, W, C),
                             lambda i: (jnp.maximum(i * hb - 1, 0), 0, 0)),
                pl.BlockSpec((2 * p, W, C),
                             lambda i: (jnp.minimum((i + 1) * hb, nblk * hb - 1), 0, 0)),
                pl.BlockSpec((k * k, C), lambda i: (0, 0))]
    args = [x, x, x, wf]
    if residual:
        in_specs += [pl.BlockSpec((1, C), lambda i: (0, 0))]
        args += [bias.reshape(1, C)]
    return _pcall(
        body,
        grid=(nblk,),
        in_specs=in_specs,
        out_specs=pl.BlockSpec((Rh, W, C), lambda i: (i, 0, 0)),
        out_shape=jax.ShapeDtypeStruct((H, W, C), _F32),
    )(*args)


@functools.lru_cache(maxsize=None)
def _pool_rows_mat(bh, bw, W, Kw):
    pm = np.zeros((Kw, bh * W), np.float32)
    for u in range(Kw):
        for r in range(bh):
            for c in range(bw):
                pm[u, r * W + u * bw + c] = 1.0 / (bh * bw)
    return jnp.asarray(pm)


@functools.lru_cache(maxsize=None)
def _win_avg_mat(n_win, kv):
    P = n_win * n_win
    m = kv * kv
    Kw = n_win * kv
    av = np.zeros((P, P * m), np.float32)
    for wi in range(n_win):
        for wj in range(n_win):
            for ki in range(kv):
                for kj in range(kv):
                    av[wi * n_win + wj, (wi * kv + ki) * Kw + wj * kv + kj] = 1.0 / m
    return jnp.asarray(av)


def _ln_qv_pool(x, g, b, wq, wv, bq, bv, pmx, Kh, Kw):
    HW, C = x.shape
    R = HW // Kh

    def body(x_ref, g_ref, b_ref, wq_ref, wv_ref, bq_ref, bv_ref, pm_ref,
             q_ref, v_ref, xp_ref):
        x_ = x_ref[...]
        mu = jnp.mean(x_, axis=-1, keepdims=True)
        xc = x_ - mu
        var = jnp.mean(xc * xc, axis=-1, keepdims=True)
        xn = xc * lax.rsqrt(var + 1e-5) * g_ref[...] + b_ref[...]
        q_ref[...] = jnp.dot(xn, wq_ref[...], preferred_element_type=_F32) + bq_ref[...]
        v_ref[...] = jnp.dot(xn, wv_ref[...], preferred_element_type=_F32) + bv_ref[...]
        xp_ref[0] = jnp.dot(pm_ref[...], xn, preferred_element_type=_F32)

    full = lambda shape: pl.BlockSpec(shape, lambda i: (0,) * len(shape))
    row = pl.BlockSpec((R, C), lambda i: (i, 0))
    return _pcall(
        body,
        grid=(Kh,),
        in_specs=[row, full((1, C)), full((1, C)), full((C, C)), full((C, C)),
                  full((1, C)), full((1, C)), full((Kw, R))],
        out_specs=[row, row, pl.BlockSpec((1, Kw, C), lambda i: (i, 0, 0))],
        out_shape=[jax.ShapeDtypeStruct((HW, C), _F32),
                   jax.ShapeDtypeStruct((HW, C), _F32),
                   jax.ShapeDtypeStruct((Kh, Kw, C), _F32)],
    )(x, g.reshape(1, C), b.reshape(1, C), wq, wv,
      bq.reshape(1, C), bv.reshape(1, C), pmx)


def _route_proj(xp, avg, P, wq, wk, wv, bq, bk, bv, topk):
    Pm, C = xp.shape

    Cpad = -(-C // 128) * 128

    def body(*refs):
        if avg is None:
            (xp_ref, wq_ref, wk_ref, wv_ref, bq_ref, bk_ref, bv_ref,
             kp_ref, vp_ref, idx_ref) = refs
        else:
            (xp_ref, avg_ref, wq_ref, wk_ref, wv_ref, bq_ref, bk_ref, bv_ref,
             kp_ref, vp_ref, idx_ref) = refs
        xp_ = xp_ref[...]
        kp = jnp.dot(xp_, wk_ref[...], preferred_element_type=_F32) + bk_ref[...]
        vp = jnp.dot(xp_, wv_ref[...], preferred_element_type=_F32) + bv_ref[...]
        if Cpad > C:
            z = jnp.zeros((Pm, Cpad - C), _F32)
            kp = jnp.concatenate([kp, z], axis=1)
            vp = jnp.concatenate([vp, z], axis=1)
        kp_ref[...] = kp
        vp_ref[...] = vp
        xm_ = xp_ if avg is None else jnp.dot(avg_ref[...], xp_,
                                              preferred_element_type=_F32)
        qm = jnp.dot(xm_, wq_ref[...], preferred_element_type=_F32) + bq_ref[...]
        km = jnp.dot(xm_, wk_ref[...], preferred_element_type=_F32) + bk_ref[...]
        a = lax.dot_general(qm, km, (((1,), (1,)), ((), ())),
                            preferred_element_type=_F32)
        col = lax.broadcasted_iota(jnp.int32, (P, P), 1)
        cols = []
        for _ in range(topk):
            mx = jnp.max(a, axis=1, keepdims=True)
            am = jnp.min(jnp.where(a >= mx, col, P), axis=1, keepdims=True)
            cols.append(am)
            a = jnp.where(col == am, -jnp.inf, a)
        idx_ref[...] = jnp.concatenate(cols, axis=1)

    args = [xp] + ([] if avg is None else [avg]) + [
        wq, wk, wv, bq.reshape(1, C), bk.reshape(1, C), bv.reshape(1, C)]
    return _pcall(
        body,
        out_shape=[jax.ShapeDtypeStruct((Pm, Cpad), _F32),
                   jax.ShapeDtypeStruct((Pm, Cpad), _F32),
                   jax.ShapeDtypeStruct((P, topk), jnp.int32)],
    )(*args)


def _sc_gather(kp, vp, idxr):
    N, C = kp.shape
    Npad = idxr.shape[0]
    b = Npad // _SC_WORKERS
    mesh = plsc.VectorSubcoreMesh(core_axis_name="c", subcore_axis_name="s")

    @functools.partial(
        pl.kernel, mesh=mesh,
        out_type=[jax.ShapeDtypeStruct((Npad, C), _F32),
                  jax.ShapeDtypeStruct((Npad, C), _F32)],
        scratch_types=[pltpu.VMEM((b,), jnp.int32),
                       pltpu.VMEM((b, C), _F32),
                       pltpu.VMEM((b, C), _F32),
                       pltpu.SemaphoreType.DMA,
                       pltpu.SemaphoreType.DMA],
    )
    def gk(kp_hbm, vp_hbm, idx_hbm, ko_hbm, vo_hbm, idx_v, kr, vr, s1, s2):
        wid = jax.lax.axis_index("s") * _SC_CORES + jax.lax.axis_index("c")
        base = wid * b
        pltpu.sync_copy(idx_hbm.at[pl.ds(base, b)], idx_v)
        for off in range(0, b, 128):
            n = min(128, b - off)
            c1 = pltpu.async_copy(kp_hbm.at[idx_v.at[pl.ds(off, n)]],
                                  kr.at[pl.ds(off, n)], s1)
            c2 = pltpu.async_copy(vp_hbm.at[idx_v.at[pl.ds(off, n)]],
                                  vr.at[pl.ds(off, n)], s2)
            c1.wait()
            c2.wait()
        pltpu.sync_copy(kr, ko_hbm.at[pl.ds(base, b)])
        pltpu.sync_copy(vr, vo_hbm.at[pl.ds(base, b)])

    return gk(kp, vp, idxr)


def _attn(qw, ks, vs, nh, kvtot):
    P, hw, C = qw.shape
    Cp = ks.shape[2]
    c = C // nh
    scale = c ** -0.5
    G = 14 if P % 14 == 0 else (7 if P % 7 == 0 else 1)

    def body(qw_ref, ks_ref, vs_ref, ow_ref):
        for g in range(G):
            q = qw_ref[g]
            ks_ = ks_ref[g]
            vs_ = vs_ref[g]
            outs = []
            for hh in range(nh):
                qh = q[:, hh * c:(hh + 1) * c] * scale
                kh = ks_[:, hh * c:(hh + 1) * c]
                vh = vs_[:, hh * c:(hh + 1) * c]
                logits = lax.dot_general(qh, kh, (((1,), (1,)), ((), ())),
                                         preferred_element_type=_F32)
                pa = jax.nn.softmax(logits, axis=-1)
                outs.append(jnp.dot(pa, vh, preferred_element_type=_F32))
            ow_ref[g] = jnp.concatenate(outs, 1) if nh > 1 else outs[0]

    return _pcall(
        body,
        grid=(P // G,),
        in_specs=[pl.BlockSpec((G, hw, C), lambda i: (i, 0, 0)),
                  pl.BlockSpec((G, kvtot, Cp), lambda i: (i, 0, 0)),
                  pl.BlockSpec((G, kvtot, Cp), lambda i: (i, 0, 0))],
        out_specs=pl.BlockSpec((G, hw, C), lambda i: (i, 0, 0)),
        out_shape=jax.ShapeDtypeStruct((P, hw, C), _F32),
    )(qw, ks, vs)


def _wo_mlp(xres, aw, lepe, wo, wob, g, b, w1, b1, w2, b2):
    HW, C = xres.shape
    C2 = w1.shape[1]
    R = _rows(HW, C)

    def body(x_ref, a_ref, l_ref, wo_ref, wob_ref, g_ref, b_ref,
             w1_ref, b1_ref, w2_ref, b2_ref, o_ref):
        y = a_ref[...] + l_ref[...]
        x1 = x_ref[...] + jnp.dot(y, wo_ref[...],
                                  preferred_element_type=_F32) + wob_ref[...]
        mu = jnp.mean(x1, axis=-1, keepdims=True)
        xc = x1 - mu
        var = jnp.mean(xc * xc, axis=-1, keepdims=True)
        xn = xc * lax.rsqrt(var + 1e-5) * g_ref[...] + b_ref[...]
        h = jax.nn.gelu(jnp.dot(xn, w1_ref[...], preferred_element_type=_F32)
                        + b1_ref[...])
        o_ref[...] = x1 + jnp.dot(h, w2_ref[...],
                                  preferred_element_type=_F32) + b2_ref[...]

    full = lambda shape: pl.BlockSpec(shape, lambda i: (0,) * len(shape))
    row = pl.BlockSpec((R, C), lambda i: (i, 0))
    return _pcall(
        body,
        grid=(HW // R,),
        in_specs=[row, row, row, full((C, C)), full((1, C)), full((1, C)),
                  full((1, C)), full((C, C2)), full((1, C2)), full((C2, C)),
                  full((1, C))],
        out_specs=row,
        out_shape=jax.ShapeDtypeStruct((HW, C), _F32),
    )(xres, aw, lepe, wo, wob.reshape(1, C), g.reshape(1, C), b.reshape(1, C),
      w1, b1.reshape(1, C2), w2, b2.reshape(1, C))


def _block(x, p, n_win, nh, topk, kv_per_win):
    C = x.shape[-1]
    x = _dwconv(x[0], p['cpe_w'], 3, bias=p['cpe_b'], residual=True)[None]
    _, H, W, _ = x.shape
    h, w = H // n_win, W // n_win
    P = n_win * n_win
    m = kv_per_win * kv_per_win
    bh, bw = h // kv_per_win, w // kv_per_win
    HW = H * W

    wq, wk, wv = jnp.split(p['qkv_w'], 3, axis=1)
    bq, bk, bv = jnp.split(p['qkv_b'], 3)

    Kh = Kw = n_win * kv_per_win
    x2 = x[0]
    pmx = _pool_rows_mat(bh, bw, W, Kw)
    q, v, xp3 = _ln_qv_pool(x2.reshape(HW, C), p['ln1_g'], p['ln1_b'],
                            wq, wv, bq, bv, pmx, Kh, Kw)
    xp = xp3.reshape(Kh * Kw, C)

    avg = _win_avg_mat(n_win, kv_per_win) if kv_per_win > 1 else None
    kp, vp, idx = _route_proj(xp, avg, P, wq, wk, wv, bq, bk, bv, topk)

    kvtot = topk * m
    kv_ = kv_per_win
    wi = idx // n_win
    wj = idx % n_win
    ki = jnp.arange(kv_, dtype=jnp.int32).reshape(1, 1, kv_, 1)
    kj = jnp.arange(kv_, dtype=jnp.int32).reshape(1, 1, 1, kv_)
    rows = ((wi[:, :, None, None] * kv_ + ki) * Kw
            + wj[:, :, None, None] * kv_ + kj).reshape(-1)
    npad = -(P * kvtot) // 256 * -256
    idxr = jnp.zeros((npad,), jnp.int32).at[:P * kvtot].set(rows)

    lepe = _dwconv(v.reshape(H, W, C), p['lepe_w'], 5).reshape(HW, C)
    qw = _win(q.reshape(H, W, C), n_win, h, w, C)

    ks, vs = _sc_gather(kp, vp, idxr)
    Cpad = kp.shape[1]
    ks = ks[:P * kvtot].reshape(P, kvtot, Cpad)
    vs = vs[:P * kvtot].reshape(P, kvtot, Cpad)

    aw = _attn(qw, ks, vs, nh, kvtot)
    a_spat = _unwin(aw, n_win, h, w, C).reshape(HW, C)

    out = _wo_mlp(x2.reshape(HW, C), a_spat, lepe, p['wo_w'], p['wo_b'],
                  p['ln2_g'], p['ln2_b'], p['mlp_w1'], p['mlp_b1'],
                  p['mlp_w2'], p['mlp_b2'])
    return out.reshape(1, H, W, C)


def kernel(img0, img1, warped_img0, warped_img1, mask, flow, c0_0, c0_1,
           c0_2, c0_3, c1_0, c1_1, c1_2, c1_3, mask_guide_0, mask_guide_1,
           mask_guide_2, params):
    p = params
    nhwc = lambda t: t.transpose(0, 2, 3, 1)
    inp = jnp.concatenate([img0, img1, mask, mask_guide_0, warped_img0,
                           warped_img1, c0_0, c1_0, flow], 1)
    x1 = _conv(nhwc(inp), p['patch_embed'])
    for bp in p['enc1']:
        x1 = _block(x1, bp, 14, 1, 6, 2)
    f1 = _conv(nhwc(jnp.concatenate([mask_guide_1, c0_1, c1_1], 1)),
               p['c_down1'])
    x2 = _pixel_unshuffle(_conv(x1, p['down1_2']), 2)
    x2 = jnp.concatenate([x2, f1], -1)
    for bp in p['enc2']:
        x2 = _block(x2, bp, 7, 2, 4, 1)
    f2 = _conv(nhwc(jnp.concatenate([mask_guide_2, c0_2, c1_2], 1)),
               p['c_down2'])
    x3 = _pixel_unshuffle(_conv(x2, p['down2_3']), 2)
    x3 = jnp.concatenate([x3, f2], -1)
    for bp in p['enc3']:
        x3 = _block(x3, bp, 7, 4, 4, 1)
    f3 = _conv(nhwc(jnp.concatenate([c0_3, c1_3], 1)), p['c_down3'])
    x4 = _pixel_unshuffle(_conv(x3, p['down3_4']), 2)
    x4 = jnp.concatenate([x4, f3], -1)
    for bp in p['latent']:
        x4 = _block(x4, bp, 7, 8, 4, 1)
    d3 = _pixel_shuffle(_conv(x4, p['up4_3']), 2)
    d3 = _conv(jnp.concatenate([d3, x3], -1), p['reduce3'])
    for bp in p['dec3']:
        d3 = _block(d3, bp, 7, 1, 4, 1)
    d2 = _pixel_shuffle(_conv(d3, p['up3_2']), 2)
    d2 = _conv(jnp.concatenate([d2, x2], -1), p['reduce2'])
    for bp in p['dec2']:
        d2 = _block(d2, bp, 7, 1, 4, 1)
    d1 = _pixel_shuffle(_conv(d2, p['up2_1']), 2)
    d1 = jnp.concatenate([d1, x1], -1)
    for bp in p['dec1']:
        d1 = _block(d1, bp, 7, 1, 4, 2)
    for bp in p['refine']:
        d1 = _block(d1, bp, 14, 1, 6, 2)
    return jax.nn.sigmoid(_conv(d1, p['out_w'])).transpose(0, 3, 1, 2)
